# Initial kernel scaffold; baseline (speedup 1.0000x reference)
import jax
import jax.numpy as jnp
from jax.experimental import pallas as pl


def kernel(input_ids, token_type_ids, item_position_ids, word_emb, pos_emb, type_emb, item_emb, ln_gamma, ln_beta):
    raise NotImplementedError("write your pallas kernel here")


# speedup vs baseline: 1.0704x; 1.0704x over previous
import functools

import jax
import jax.numpy as jnp
from jax import lax
from jax.experimental import pallas as pl
from jax.experimental.pallas import tpu as pltpu
from jax.experimental.pallas import tpu_sc as plsc

_VOCAB = 1000000
_HID = 64
_PAD = 1
_B = 1024
_L = 200
_EPS = 1e-12

_NC = 2
_NS = 16
_NW = _NC * _NS
_TOK = _B * _L
_TPW = _TOK // _NW
_RPW = _B // _NW
_CH = 128
_NCH = _TPW // _CH


def _rsqrt16(x):
    xi = plsc.bitcast(x, jnp.int32)
    yi = 0x5F3759DF - lax.shift_right_logical(xi, 1)
    y = plsc.bitcast(yi, jnp.float32)
    half_x = 0.5 * x
    for _ in range(4):
        y = y * (1.5 - half_x * y * y)
    return y


def _sc_body(ids_hbm, tt_hbm, it_hbm, word_hbm, pos_hbm, type_hbm, item_hbm,
             g_hbm, b_hbm, out_hbm,
             ids_v, tt_v, it_v, pid_v, type_v, item_v, g_v, b_v,
             w_buf, p_buf, e_v, m_buf, i_buf, out_v, sem_w, sem_p):
    i32 = jnp.int32
    wid = lax.axis_index("s") * _NC + lax.axis_index("c")
    base = wid * _TPW

    pltpu.sync_copy(ids_hbm.at[pl.ds(base, _TPW)], ids_v)
    pltpu.sync_copy(tt_hbm.at[pl.ds(base, _TPW)], tt_v)
    pltpu.sync_copy(it_hbm.at[pl.ds(base, _TPW)], it_v)
    pltpu.sync_copy(type_hbm, type_v)
    pltpu.sync_copy(item_hbm, item_v)
    pltpu.sync_copy(g_hbm, g_v)
    pltpu.sync_copy(b_hbm, b_v)

    lanes = lax.iota(i32, 16)

    for g2 in range(_RPW // 16):
        row_off = lanes * _L + g2 * 16 * _L
        def cum_body(l, cum, row_off=row_off):
            idv = plsc.load_gather(ids_v, [row_off + l])
            m = jnp.where(idv != _PAD, 1, 0).astype(i32)
            cum = cum + m
            plsc.store_scatter(pid_v, [row_off + l], cum * m + _PAD)
            return cum
        lax.fori_loop(0, _L, cum_body, jnp.zeros((16,), i32))

    gvecs = [g_v[pl.ds(j * 16, 16)] for j in range(4)]
    bvecs = [b_v[pl.ds(j * 16, 16)] for j in range(4)]
    zero16f = jnp.zeros((16,), jnp.float32)
    zero16i = jnp.zeros((16,), i32)

    def chunk_body(c, _):
        tok0 = c * _CH
        cw = pltpu.async_copy(word_hbm.at[ids_v.at[pl.ds(tok0, _CH)]], w_buf,
                              sem_w)
        cp = pltpu.async_copy(pos_hbm.at[pid_v.at[pl.ds(tok0, _CH)]], p_buf,
                              sem_p)
        cw.wait()
        cp.wait()

        for g in range(_CH // 16):
            tok16 = lanes + g * 16
            ttv = tt_v[pl.ds(tok0 + g * 16, 16)] * _HID
            itv = it_v[pl.ds(tok0 + g * 16, 16)] * _HID
            e_base = tok16 * _HID

            def h_body(h, carry, tok16=tok16, ttv=ttv, itv=itv, e_base=e_base):
                s, q = carry
                hsp = zero16i + h
                w = plsc.load_gather(w_buf, [tok16, hsp])
                p = plsc.load_gather(p_buf, [tok16, hsp])
                t = plsc.load_gather(type_v, [ttv + h])
                it = plsc.load_gather(item_v, [itv + h])
                e = (w + p) + (t + it)
                plsc.store_scatter(e_v, [e_base + h], e)
                return (s + e, q + e * e)

            s, q = lax.fori_loop(0, _HID, h_body, (zero16f, zero16f))
            mean = s * (1.0 / _HID)
            var = q * (1.0 / _HID) - mean * mean
            m_buf[pl.ds(g * 16, 16)] = mean
            i_buf[pl.ds(g * 16, 16)] = _rsqrt16(var + _EPS)

        def t_body(t, _):
            tsp = zero16i + t
            mean = plsc.load_gather(m_buf, [tsp])
            inv = plsc.load_gather(i_buf, [tsp])
            for j in range(4):
                e = e_v[pl.ds(t * _HID + j * 16, 16)]
                out_v[pl.ds(t * _HID + j * 16, 16)] = (
                    (e - mean) * inv * gvecs[j] + bvecs[j])
            return 0

        lax.fori_loop(0, _CH, t_body, 0)

        pltpu.sync_copy(out_v, out_hbm.at[pl.ds((base + tok0) * _HID,
                                                _CH * _HID)])
        return 0

    lax.fori_loop(0, _NCH, chunk_body, 0)


@functools.partial(jax.jit, static_argnames=())
def kernel(input_ids, token_type_ids, item_position_ids, word_emb, pos_emb,
           type_emb, item_emb, ln_gamma, ln_beta):
    ids = input_ids.reshape(-1).astype(jnp.int32)
    tt = token_type_ids.reshape(-1).astype(jnp.int32)
    it = item_position_ids.reshape(-1).astype(jnp.int32)

    k = pl.kernel(
        _sc_body,
        mesh=plsc.VectorSubcoreMesh(core_axis_name="c", subcore_axis_name="s"),
        compiler_params=pltpu.CompilerParams(needs_layout_passes=False,
                                             use_tc_tiling_on_sc=False),
        out_type=jax.ShapeDtypeStruct((_TOK * _HID,), jnp.float32),
        scratch_types=[
            pltpu.VMEM((_TPW,), jnp.int32),
            pltpu.VMEM((_TPW,), jnp.int32),
            pltpu.VMEM((_TPW,), jnp.int32),
            pltpu.VMEM((_TPW,), jnp.int32),
            pltpu.VMEM((4 * _HID,), jnp.float32),
            pltpu.VMEM((32 * _HID,), jnp.float32),
            pltpu.VMEM((_HID,), jnp.float32),
            pltpu.VMEM((_HID,), jnp.float32),
            pltpu.VMEM((_CH, _HID), jnp.float32),
            pltpu.VMEM((_CH, _HID), jnp.float32),
            pltpu.VMEM((_CH * _HID,), jnp.float32),
            pltpu.VMEM((_CH,), jnp.float32),
            pltpu.VMEM((_CH,), jnp.float32),
            pltpu.VMEM((_CH * _HID,), jnp.float32),
            pltpu.SemaphoreType.DMA,
            pltpu.SemaphoreType.DMA,
        ],
    )
    flat = k(ids, tt, it, word_emb, pos_emb, type_emb.reshape(-1),
             item_emb.reshape(-1), ln_gamma, ln_beta)
    return flat.reshape(_B, _L, _HID)


# speedup vs baseline: 1.5395x; 1.4383x over previous
import functools

import jax
import jax.numpy as jnp
from jax import lax
from jax.experimental import pallas as pl
from jax.experimental.pallas import tpu as pltpu
from jax.experimental.pallas import tpu_sc as plsc

_VOCAB = 1000000
_HID = 64
_PAD = 1
_B = 1024
_L = 200
_EPS = 1e-12

_NC = 2
_NS = 16
_NW = _NC * _NS
_TOK = _B * _L
_TPW = _TOK // _NW
_RPW = _B // _NW
_CH = 128
_NCH = _TPW // _CH


def _rsqrt16(x):
    xi = plsc.bitcast(x, jnp.int32)
    yi = 0x5F3759DF - lax.shift_right_logical(xi, 1)
    y = plsc.bitcast(yi, jnp.float32)
    half_x = 0.5 * x
    for _ in range(4):
        y = y * (1.5 - half_x * y * y)
    return y


def _sc_body(ids_hbm, tt_hbm, it_hbm, word_hbm, pos_hbm, type_hbm, item_hbm,
             g_hbm, b_hbm, out_hbm,
             ids_v, tt_v, it_v, pid_v, type_v, item_v, g_v, b_v,
             w0, p0, w1, p1, e_v, m_buf, i_buf, out_v,
             sem_w0, sem_p0, sem_w1, sem_p1):
    i32 = jnp.int32
    wid = lax.axis_index("s") * _NC + lax.axis_index("c")
    base = wid * _TPW

    pltpu.sync_copy(ids_hbm.at[pl.ds(base, _TPW)], ids_v)
    pltpu.sync_copy(tt_hbm.at[pl.ds(base, _TPW)], tt_v)
    pltpu.sync_copy(it_hbm.at[pl.ds(base, _TPW)], it_v)
    pltpu.sync_copy(type_hbm, type_v)
    pltpu.sync_copy(item_hbm, item_v)
    pltpu.sync_copy(g_hbm, g_v)
    pltpu.sync_copy(b_hbm, b_v)

    lanes = lax.iota(i32, 16)

    for g2 in range(_RPW // 16):
        row_off = lanes * _L + g2 * 16 * _L

        def cum_body(l, cum, row_off=row_off):
            idv = plsc.load_gather(ids_v, [row_off + l])
            m = jnp.where(idv != _PAD, 1, 0).astype(i32)
            cum = cum + m
            plsc.store_scatter(pid_v, [row_off + l], cum * m + _PAD)
            return cum
        plsc.parallel_loop(0, _L, unroll=8,
                           carry=jnp.zeros((16,), i32))(cum_body)

    gvecs = [g_v[pl.ds(j * 16, 16)] for j in range(4)]
    bvecs = [b_v[pl.ds(j * 16, 16)] for j in range(4)]
    zero16f = jnp.zeros((16,), jnp.float32)
    zero16i = jnp.zeros((16,), i32)

    def issue(c, wb, pb, sw, sp):
        cw = pltpu.async_copy(word_hbm.at[ids_v.at[pl.ds(c * _CH, _CH)]],
                              wb, sw)
        cp = pltpu.async_copy(pos_hbm.at[pid_v.at[pl.ds(c * _CH, _CH)]],
                              pb, sp)
        return cw, cp

    def compute(c, wb, pb):
        tok0 = c * _CH
        for g in range(_CH // 16):
            tok16 = lanes + g * 16
            ttv = tt_v[pl.ds(tok0 + g * 16, 16)] * _HID
            itv = it_v[pl.ds(tok0 + g * 16, 16)] * _HID
            e_base = tok16 * _HID

            def h_body(h, carry, tok16=tok16, ttv=ttv, itv=itv,
                       e_base=e_base):
                s, q = carry
                eb = e_base + h
                hsp = zero16i + h
                w = plsc.load_gather(wb, [tok16, hsp])
                p = plsc.load_gather(pb, [tok16, hsp])
                t = plsc.load_gather(type_v, [ttv + h])
                it = plsc.load_gather(item_v, [itv + h])
                e = (w + p) + (t + it)
                plsc.store_scatter(e_v, [eb], e)
                return (s + e, q + e * e)

            s, q = plsc.parallel_loop(0, _HID, unroll=8,
                                      carry=(zero16f, zero16f))(h_body)
            mean = s * (1.0 / _HID)
            var = q * (1.0 / _HID) - mean * mean
            m_buf[pl.ds(g * 16, 16)] = mean
            i_buf[pl.ds(g * 16, 16)] = _rsqrt16(var + _EPS)

        def t_body(t):
            tsp = zero16i + t
            mean = plsc.load_gather(m_buf, [tsp])
            inv = plsc.load_gather(i_buf, [tsp])
            for j in range(4):
                e = e_v[pl.ds(t * _HID + j * 16, 16)]
                out_v[pl.ds(t * _HID + j * 16, 16)] = (
                    (e - mean) * inv * gvecs[j] + bvecs[j])
        plsc.parallel_loop(0, _CH, unroll=4)(t_body)

        pltpu.sync_copy(out_v, out_hbm.at[pl.ds((base + tok0) * _HID,
                                                _CH * _HID)])

    issue(0, w0, p0, sem_w0, sem_p0)

    def drain(buf, sem):
        pltpu.make_async_copy(out_hbm.at[pl.ds(0, _CH * _HID)], buf,
                              sem).wait()

    def body2(c2, _):
        c = c2 * 2
        drain(w0, sem_w0)
        drain(p0, sem_p0)
        issue(c + 1, w1, p1, sem_w1, sem_p1)
        compute(c, w0, p0)
        drain(w1, sem_w1)
        drain(p1, sem_p1)
        nxt = jnp.minimum(c + 2, _NCH - 2)
        issue(nxt, w0, p0, sem_w0, sem_p0)
        compute(c + 1, w1, p1)
        return 0

    lax.fori_loop(0, _NCH // 2, body2, 0)
    drain(w0, sem_w0)
    drain(p0, sem_p0)


@jax.jit
def kernel(input_ids, token_type_ids, item_position_ids, word_emb, pos_emb,
           type_emb, item_emb, ln_gamma, ln_beta):
    ids = input_ids.reshape(-1).astype(jnp.int32)
    tt = token_type_ids.reshape(-1).astype(jnp.int32)
    it = item_position_ids.reshape(-1).astype(jnp.int32)

    k = pl.kernel(
        _sc_body,
        mesh=plsc.VectorSubcoreMesh(core_axis_name="c", subcore_axis_name="s"),
        compiler_params=pltpu.CompilerParams(needs_layout_passes=False,
                                             use_tc_tiling_on_sc=False),
        out_type=jax.ShapeDtypeStruct((_TOK * _HID,), jnp.float32),
        scratch_types=[
            pltpu.VMEM((_TPW,), jnp.int32),
            pltpu.VMEM((_TPW,), jnp.int32),
            pltpu.VMEM((_TPW,), jnp.int32),
            pltpu.VMEM((_TPW,), jnp.int32),
            pltpu.VMEM((4 * _HID,), jnp.float32),
            pltpu.VMEM((32 * _HID,), jnp.float32),
            pltpu.VMEM((_HID,), jnp.float32),
            pltpu.VMEM((_HID,), jnp.float32),
            pltpu.VMEM((_CH, _HID), jnp.float32),
            pltpu.VMEM((_CH, _HID), jnp.float32),
            pltpu.VMEM((_CH, _HID), jnp.float32),
            pltpu.VMEM((_CH, _HID), jnp.float32),
            pltpu.VMEM((_CH * _HID,), jnp.float32),
            pltpu.VMEM((_CH,), jnp.float32),
            pltpu.VMEM((_CH,), jnp.float32),
            pltpu.VMEM((_CH * _HID,), jnp.float32),
            pltpu.SemaphoreType.DMA,
            pltpu.SemaphoreType.DMA,
            pltpu.SemaphoreType.DMA,
            pltpu.SemaphoreType.DMA,
        ],
    )
    flat = k(ids, tt, it, word_emb, pos_emb, type_emb.reshape(-1),
             item_emb.reshape(-1), ln_gamma, ln_beta)
    return flat.reshape(_B, _L, _HID)


# speedup vs baseline: 1.6699x; 1.0847x over previous
import functools

import jax
import jax.numpy as jnp
from jax import lax
from jax.experimental import pallas as pl
from jax.experimental.pallas import tpu as pltpu
from jax.experimental.pallas import tpu_sc as plsc

_VOCAB = 1000000
_HID = 64
_PAD = 1
_B = 1024
_L = 200
_EPS = 1e-12

_NC = 2
_NS = 16
_NW = _NC * _NS
_TOK = _B * _L
_TPW = _TOK // _NW
_RPW = _B // _NW
_CH = 128
_NCH = _TPW // _CH


def _rsqrt16(x):
    xi = plsc.bitcast(x, jnp.int32)
    yi = 0x5F3759DF - lax.shift_right_logical(xi, 1)
    y = plsc.bitcast(yi, jnp.float32)
    half_x = 0.5 * x
    for _ in range(4):
        y = y * (1.5 - half_x * y * y)
    return y


def _sc_body(ids_hbm, tt_hbm, it_hbm, word_hbm, pos_hbm, type_hbm, item_hbm,
             g_hbm, b_hbm, out_hbm,
             ids_v, tt_v, it_v, pid_v, type_v, item_v, g_v, b_v,
             w0, p0, w1, p1, out_v,
             sem_w0, sem_p0, sem_w1, sem_p1):
    i32 = jnp.int32
    wid = lax.axis_index("s") * _NC + lax.axis_index("c")
    base = wid * _TPW

    pltpu.sync_copy(ids_hbm.at[pl.ds(base, _TPW)], ids_v)
    pltpu.sync_copy(tt_hbm.at[pl.ds(base, _TPW)], tt_v)
    pltpu.sync_copy(it_hbm.at[pl.ds(base, _TPW)], it_v)
    pltpu.sync_copy(type_hbm, type_v)
    pltpu.sync_copy(item_hbm, item_v)
    pltpu.sync_copy(g_hbm, g_v)
    pltpu.sync_copy(b_hbm, b_v)

    lanes = lax.iota(i32, 16)

    for g2 in range(_RPW // 16):
        row_off = lanes * _L + g2 * 16 * _L

        def cum_body(l, cum, row_off=row_off):
            idv = plsc.load_gather(ids_v, [row_off + l])
            m = jnp.where(idv != _PAD, 1, 0).astype(i32)
            cum = cum + m
            plsc.store_scatter(pid_v, [row_off + l], cum * m + _PAD)
            return cum
        plsc.parallel_loop(0, _L, unroll=8,
                           carry=jnp.zeros((16,), i32))(cum_body)

    gvecs = [g_v[pl.ds(j * 16, 16)] for j in range(4)]
    bvecs = [b_v[pl.ds(j * 16, 16)] for j in range(4)]
    zero16f = jnp.zeros((16,), jnp.float32)
    zero16i = jnp.zeros((16,), i32)

    def issue(c, wb, pb, sw, sp):
        cw = pltpu.async_copy(word_hbm.at[ids_v.at[pl.ds(c * _CH, _CH)]],
                              wb, sw)
        cp = pltpu.async_copy(pos_hbm.at[pid_v.at[pl.ds(c * _CH, _CH)]],
                              pb, sp)
        return cw, cp

    def compute(c, wb, pb):
        tok0 = c * _CH

        def g_body(g):
            ttv = tt_v[pl.ds(tok0 + g * 16, 16)] * _HID
            itv = it_v[pl.ds(tok0 + g * 16, 16)] * _HID
            for k in range(16):
                t = g * 16 + k
                tt = ttv[k]
                it = itv[k]
                e = [wb[t, pl.ds(j * 16, 16)] + pb[t, pl.ds(j * 16, 16)]
                     + type_v[pl.ds(tt + j * 16, 16)]
                     + item_v[pl.ds(it + j * 16, 16)]
                     for j in range(4)]
                s = (e[0] + e[1]) + (e[2] + e[3])
                q = ((e[0] * e[0] + e[1] * e[1])
                     + (e[2] * e[2] + e[3] * e[3]))
                tot = jnp.sum(s, axis=0)
                totq = jnp.sum(q, axis=0)
                mean = jnp.full((16,), tot, jnp.float32) * (1.0 / _HID)
                var = (jnp.full((16,), totq, jnp.float32) * (1.0 / _HID)
                       - mean * mean)
                inv = _rsqrt16(var + _EPS)
                for j in range(4):
                    out_v[pl.ds(t * _HID + j * 16, 16)] = (
                        (e[j] - mean) * inv * gvecs[j] + bvecs[j])
        plsc.parallel_loop(0, _CH // 16, unroll=2)(g_body)

        pltpu.sync_copy(out_v, out_hbm.at[pl.ds((base + tok0) * _HID,
                                                _CH * _HID)])

    issue(0, w0, p0, sem_w0, sem_p0)

    def drain(buf, sem):
        pltpu.make_async_copy(out_hbm.at[pl.ds(0, _CH * _HID)], buf,
                              sem).wait()

    def body2(c2, _):
        c = c2 * 2
        drain(w0, sem_w0)
        drain(p0, sem_p0)
        issue(c + 1, w1, p1, sem_w1, sem_p1)
        compute(c, w0, p0)
        drain(w1, sem_w1)
        drain(p1, sem_p1)
        nxt = jnp.minimum(c + 2, _NCH - 2)
        issue(nxt, w0, p0, sem_w0, sem_p0)
        compute(c + 1, w1, p1)
        return 0

    lax.fori_loop(0, _NCH // 2, body2, 0)
    drain(w0, sem_w0)
    drain(p0, sem_p0)


@jax.jit
def kernel(input_ids, token_type_ids, item_position_ids, word_emb, pos_emb,
           type_emb, item_emb, ln_gamma, ln_beta):
    ids = input_ids.reshape(-1).astype(jnp.int32)
    tt = token_type_ids.reshape(-1).astype(jnp.int32)
    it = item_position_ids.reshape(-1).astype(jnp.int32)

    k = pl.kernel(
        _sc_body,
        mesh=plsc.VectorSubcoreMesh(core_axis_name="c", subcore_axis_name="s"),
        compiler_params=pltpu.CompilerParams(needs_layout_passes=False,
                                             use_tc_tiling_on_sc=False),
        out_type=jax.ShapeDtypeStruct((_TOK * _HID,), jnp.float32),
        scratch_types=[
            pltpu.VMEM((_TPW,), jnp.int32),
            pltpu.VMEM((_TPW,), jnp.int32),
            pltpu.VMEM((_TPW,), jnp.int32),
            pltpu.VMEM((_TPW,), jnp.int32),
            pltpu.VMEM((4 * _HID,), jnp.float32),
            pltpu.VMEM((32 * _HID,), jnp.float32),
            pltpu.VMEM((_HID,), jnp.float32),
            pltpu.VMEM((_HID,), jnp.float32),
            pltpu.VMEM((_CH, _HID), jnp.float32),
            pltpu.VMEM((_CH, _HID), jnp.float32),
            pltpu.VMEM((_CH, _HID), jnp.float32),
            pltpu.VMEM((_CH, _HID), jnp.float32),
            pltpu.VMEM((_CH * _HID,), jnp.float32),
            pltpu.SemaphoreType.DMA,
            pltpu.SemaphoreType.DMA,
            pltpu.SemaphoreType.DMA,
            pltpu.SemaphoreType.DMA,
        ],
    )
    flat = k(ids, tt, it, word_emb, pos_emb, type_emb.reshape(-1),
             item_emb.reshape(-1), ln_gamma, ln_beta)
    return flat.reshape(_B, _L, _HID)


# speedup vs baseline: 2.5627x; 1.5346x over previous
import functools

import jax
import jax.numpy as jnp
from jax import lax
from jax.experimental import pallas as pl
from jax.experimental.pallas import tpu as pltpu
from jax.experimental.pallas import tpu_sc as plsc

_VOCAB = 1000000
_HID = 64
_PAD = 1
_B = 1024
_L = 200
_EPS = 1e-12

_NC = 2
_NS = 16
_NW = _NC * _NS
_TOK = _B * _L
_TPW = _TOK // _NW
_RPW = _B // _NW
_CH = 128
_NCH = _TPW // _CH


def _rsqrt16(x):
    xi = plsc.bitcast(x, jnp.int32)
    yi = 0x5F3759DF - lax.shift_right_logical(xi, 1)
    y = plsc.bitcast(yi, jnp.float32)
    half_x = 0.5 * x
    for _ in range(4):
        y = y * (1.5 - half_x * y * y)
    return y


def _sc_body(ids_hbm, tt_hbm, it_hbm, word_hbm, pos_hbm, type_hbm, item_hbm,
             g_hbm, b_hbm, out_hbm,
             ids_v, tt_v, it_v, pid_v, type_v, item_v, g_v, b_v,
             w0, p0, w1, p1, ev, sb, qb, mb, ib, out_v,
             sem_w0, sem_p0, sem_w1, sem_p1):
    i32 = jnp.int32
    wid = lax.axis_index("s") * _NC + lax.axis_index("c")
    base = wid * _TPW

    pltpu.sync_copy(ids_hbm.at[pl.ds(base, _TPW)], ids_v)
    pltpu.sync_copy(tt_hbm.at[pl.ds(base, _TPW)], tt_v)
    pltpu.sync_copy(it_hbm.at[pl.ds(base, _TPW)], it_v)
    pltpu.sync_copy(type_hbm, type_v)
    pltpu.sync_copy(item_hbm, item_v)
    pltpu.sync_copy(g_hbm, g_v)
    pltpu.sync_copy(b_hbm, b_v)

    lanes = lax.iota(i32, 16)

    for g2 in range(_RPW // 16):
        row_off = lanes * _L + g2 * 16 * _L

        def cum_body(l, cum, row_off=row_off):
            idv = plsc.load_gather(ids_v, [row_off + l])
            m = jnp.where(idv != _PAD, 1, 0).astype(i32)
            cum = cum + m
            plsc.store_scatter(pid_v, [row_off + l], cum * m + _PAD)
            return cum
        plsc.parallel_loop(0, _L, unroll=8,
                           carry=jnp.zeros((16,), i32))(cum_body)

    gvecs = [g_v[pl.ds(j * 16, 16)] for j in range(4)]
    bvecs = [b_v[pl.ds(j * 16, 16)] for j in range(4)]
    zero16f = jnp.zeros((16,), jnp.float32)
    zero16i = jnp.zeros((16,), i32)

    def issue(c, wb, pb, sw, sp):
        cw = pltpu.async_copy(word_hbm.at[ids_v.at[pl.ds(c * _CH, _CH)]],
                              wb, sw)
        cp = pltpu.async_copy(pos_hbm.at[pid_v.at[pl.ds(c * _CH, _CH)]],
                              pb, sp)
        return cw, cp

    def compute(c, wb, pb):
        tok0 = c * _CH

        def p1_body(g):
            ttv = tt_v[pl.ds(tok0 + g * 16, 16)] * _HID
            itv = it_v[pl.ds(tok0 + g * 16, 16)] * _HID
            for k in range(16):
                t = g * 16 + k
                tt = ttv[k]
                it = itv[k]
                e = [wb[t, pl.ds(j * 16, 16)] + pb[t, pl.ds(j * 16, 16)]
                     + type_v[pl.ds(tt + j * 16, 16)]
                     + item_v[pl.ds(it + j * 16, 16)]
                     for j in range(4)]
                s = (e[0] + e[1]) + (e[2] + e[3])
                q = ((e[0] * e[0] + e[1] * e[1])
                     + (e[2] * e[2] + e[3] * e[3]))
                for j in range(4):
                    ev[pl.ds(t * _HID + j * 16, 16)] = e[j]
                sb[pl.ds(t * 24, 16)] = plsc.cumsum(s)
                qb[pl.ds(t * 24, 16)] = plsc.cumsum(q)
        plsc.parallel_loop(0, _CH // 16, unroll=2)(p1_body)

        lane24 = lanes * 24 + 15
        def p2_body(g):
            sumv = plsc.load_gather(sb, [lane24 + g * (16 * 24)])
            sqv = plsc.load_gather(qb, [lane24 + g * (16 * 24)])
            mean = sumv * (1.0 / _HID)
            var = sqv * (1.0 / _HID) - mean * mean
            mb[pl.ds(g * 16, 16)] = mean
            ib[pl.ds(g * 16, 16)] = _rsqrt16(var + _EPS)
        plsc.parallel_loop(0, _CH // 16, unroll=2)(p2_body)

        def p3_body(g):
            mv = mb[pl.ds(g * 16, 16)]
            iv = ib[pl.ds(g * 16, 16)]
            for k in range(16):
                t = g * 16 + k
                mean = jnp.full((16,), mv[k], jnp.float32)
                inv = jnp.full((16,), iv[k], jnp.float32)
                for j in range(4):
                    ej = ev[pl.ds(t * _HID + j * 16, 16)]
                    out_v[pl.ds(t * _HID + j * 16, 16)] = (
                        (ej - mean) * inv * gvecs[j] + bvecs[j])
        plsc.parallel_loop(0, _CH // 16, unroll=2)(p3_body)

        pltpu.sync_copy(out_v, out_hbm.at[pl.ds((base + tok0) * _HID,
                                                _CH * _HID)])

    issue(0, w0, p0, sem_w0, sem_p0)

    def drain(buf, sem):
        pltpu.make_async_copy(out_hbm.at[pl.ds(0, _CH * _HID)], buf,
                              sem).wait()

    def body2(c2, _):
        c = c2 * 2
        drain(w0, sem_w0)
        drain(p0, sem_p0)
        issue(c + 1, w1, p1, sem_w1, sem_p1)
        compute(c, w0, p0)
        drain(w1, sem_w1)
        drain(p1, sem_p1)
        nxt = jnp.minimum(c + 2, _NCH - 2)
        issue(nxt, w0, p0, sem_w0, sem_p0)
        compute(c + 1, w1, p1)
        return 0

    lax.fori_loop(0, _NCH // 2, body2, 0)
    drain(w0, sem_w0)
    drain(p0, sem_p0)


@jax.jit
def kernel(input_ids, token_type_ids, item_position_ids, word_emb, pos_emb,
           type_emb, item_emb, ln_gamma, ln_beta):
    ids = input_ids.reshape(-1).astype(jnp.int32)
    tt = token_type_ids.reshape(-1).astype(jnp.int32)
    it = item_position_ids.reshape(-1).astype(jnp.int32)

    k = pl.kernel(
        _sc_body,
        mesh=plsc.VectorSubcoreMesh(core_axis_name="c", subcore_axis_name="s"),
        compiler_params=pltpu.CompilerParams(needs_layout_passes=False,
                                             use_tc_tiling_on_sc=False),
        out_type=jax.ShapeDtypeStruct((_TOK * _HID,), jnp.float32),
        scratch_types=[
            pltpu.VMEM((_TPW,), jnp.int32),
            pltpu.VMEM((_TPW,), jnp.int32),
            pltpu.VMEM((_TPW,), jnp.int32),
            pltpu.VMEM((_TPW,), jnp.int32),
            pltpu.VMEM((4 * _HID,), jnp.float32),
            pltpu.VMEM((32 * _HID,), jnp.float32),
            pltpu.VMEM((_HID,), jnp.float32),
            pltpu.VMEM((_HID,), jnp.float32),
            pltpu.VMEM((_CH, _HID), jnp.float32),
            pltpu.VMEM((_CH, _HID), jnp.float32),
            pltpu.VMEM((_CH, _HID), jnp.float32),
            pltpu.VMEM((_CH, _HID), jnp.float32),
            pltpu.VMEM((_CH * _HID,), jnp.float32),
            pltpu.VMEM((_CH * 24,), jnp.float32),
            pltpu.VMEM((_CH * 24,), jnp.float32),
            pltpu.VMEM((_CH,), jnp.float32),
            pltpu.VMEM((_CH,), jnp.float32),
            pltpu.VMEM((_CH * _HID,), jnp.float32),
            pltpu.SemaphoreType.DMA,
            pltpu.SemaphoreType.DMA,
            pltpu.SemaphoreType.DMA,
            pltpu.SemaphoreType.DMA,
        ],
    )
    flat = k(ids, tt, it, word_emb, pos_emb, type_emb.reshape(-1),
             item_emb.reshape(-1), ln_gamma, ln_beta)
    return flat.reshape(_B, _L, _HID)
